# baseline (speedup 1.0000x reference)
import functools

import jax
import jax.numpy as jnp
from jax import lax
from jax.experimental import pallas as pl
from jax.experimental.pallas import tpu as pltpu
from jax.experimental.pallas import tpu_sc as plsc

N_NODES = 10000
D = 128
DH = D // 2
E = 320000
NS = 16
CS = 128
NCHUNKP = 160
EPAD = NS * NCHUNKP * CS
GARB = N_NODES
ACC_ROWS = 10016
ZB = 16
WB = 80
NB = 5
NBI = 10
LA = 3


def _hop_pass(x_hbm, o_hbm, d_hbm, s_hbm, xspm, isd, iss, rbufs, zbuf, acc,
              isems, gsems, csems, s, row0):
    def xblk(j, carry):
        r0 = (s + j * NS) * WB
        pltpu.sync_copy(x_hbm.at[pl.ds(r0, WB)], xspm.at[pl.ds(r0, WB)])
        return carry

    lax.fori_loop(0, (140 - s) // 16, xblk, 0)

    def zblk(j, carry):
        pltpu.sync_copy(zbuf, acc.at[pl.ds((s + j * NS) * ZB, ZB)])
        return carry

    lax.fori_loop(0, (641 - s) // 16, zblk, 0)
    plsc.subcore_barrier()

    def fire_idx(i, t):
        pltpu.async_copy(d_hbm.at[row0 + i], isd.at[t], isems[t])
        pltpu.async_copy(s_hbm.at[row0 + i], iss.at[t], isems[t])

    def wait_idx(i, t):
        pltpu.make_async_copy(d_hbm.at[row0 + i], isd.at[t], isems[t]).wait()
        pltpu.make_async_copy(s_hbm.at[row0 + i], iss.at[t], isems[t]).wait()

    def fire_gather(i, b, t):
        pltpu.async_copy(xspm.at[iss.at[t]], rbufs[b], gsems[b])

    def wait_gather(i, b, t):
        pltpu.make_async_copy(xspm.at[iss.at[t]], rbufs[b], gsems[b]).wait()

    def fire_scatter(i, b, t):
        pltpu.async_copy(rbufs[b], acc.at[isd.at[t]], csems[b], add=True)

    def wait_scatter(i, b, t):
        pltpu.make_async_copy(rbufs[b], acc.at[isd.at[t]], csems[b]).wait()

    for i in range(2 * LA - 1):
        fire_idx(i, i)
    for i in range(LA):
        wait_idx(i, i)
        fire_gather(i, i, i)

    def do_step(i, b, skip_scatter_wait):
        @pl.when(i + LA < NCHUNKP)
        def _():
            if not skip_scatter_wait:
                wait_scatter(i - (NB - LA), (b + LA) % NB,
                             (b - (NB - LA)) % NBI)
            wait_idx(i + LA, (b + LA) % NBI)
            fire_gather(i + LA, (b + LA) % NB, (b + LA) % NBI)

        @pl.when(i + 2 * LA - 1 < NCHUNKP)
        def _():
            fire_idx(i + 2 * LA - 1, (b + 2 * LA - 1) % NBI)

        wait_gather(i, b % NB, b % NBI)
        fire_scatter(i, b % NB, b % NBI)

    for b in range(NBI):
        do_step(b, b, b < NB - LA)

    def group(g, carry):
        i0 = NBI * g
        for b in range(NBI):
            do_step(i0 + b, b, False)
        return carry

    lax.fori_loop(1, NCHUNKP // NBI, group, 0)

    for i in range(NCHUNKP - NB, NCHUNKP):
        wait_scatter(i, i % NB, i % NBI)
    plsc.subcore_barrier()

    def wblk(j, carry):
        r0 = (s + j * NS) * WB
        pltpu.sync_copy(acc.at[pl.ds(r0, WB)], o_hbm.at[pl.ds(r0, WB)])
        return carry

    lax.fori_loop(0, (140 - s) // 16, wblk, 0)
    plsc.subcore_barrier()


def _body(xa, xb, d1, s1, d2, s2, o1a, o1b, o2a, o2b,
          isd, iss, r0, r1, r2, r3, r4, zbuf, xspm, acc,
          i0, i1, i2, i3, i4, i5, i6, i7, i8, i9,
          g0, g1, g2, g3, g4, c0, c1, c2, c3, c4):
    c = lax.axis_index("c")
    s = lax.axis_index("s")
    rbufs = (r0, r1, r2, r3, r4)
    isems = (i0, i1, i2, i3, i4, i5, i6, i7, i8, i9)
    gsems = (g0, g1, g2, g3, g4)
    csems = (c0, c1, c2, c3, c4)
    row0 = s * NCHUNKP

    def zrow(i, carry):
        for k in range(DH // 16):
            zbuf[i, pl.ds(k * 16, 16)] = jnp.zeros((16,), jnp.float32)
        return carry

    lax.fori_loop(0, ZB, zrow, 0)

    @pl.when(c == 0)
    def _():
        _hop_pass(xa, o1a, d1, s1, xspm, isd, iss, rbufs, zbuf, acc,
                  isems, gsems, csems, s, row0)
        _hop_pass(xb, o1b, d1, s1, xspm, isd, iss, rbufs, zbuf, acc,
                  isems, gsems, csems, s, row0)

    @pl.when(c == 1)
    def _():
        _hop_pass(xa, o2a, d2, s2, xspm, isd, iss, rbufs, zbuf, acc,
                  isems, gsems, csems, s, row0)
        _hop_pass(xb, o2b, d2, s2, xspm, isd, iss, rbufs, zbuf, acc,
                  isems, gsems, csems, s, row0)


_half_out = jax.ShapeDtypeStruct((N_NODES, DH), jnp.float32)

_sc_kernel = functools.partial(
    pl.kernel,
    mesh=plsc.VectorSubcoreMesh(core_axis_name="c", subcore_axis_name="s"),
    out_type=[_half_out, _half_out, _half_out, _half_out],
    compiler_params=pltpu.CompilerParams(use_tc_tiling_on_sc=False),
    scratch_types=[
        pltpu.VMEM((NBI, CS), jnp.int32),
        pltpu.VMEM((NBI, CS), jnp.int32),
    ] + [pltpu.VMEM((CS, DH), jnp.float32)] * NB
    + [
        pltpu.VMEM((ZB, DH), jnp.float32),
        pltpu.VMEM_SHARED((N_NODES, DH), jnp.float32),
        pltpu.VMEM_SHARED((ACC_ROWS, DH), jnp.float32),
    ] + [pltpu.SemaphoreType.DMA] * (NBI + 2 * NB),
)(_body)


def _pad_idx(row, pad_base, pad_mod):
    pad = pad_base + jnp.arange(EPAD - E, dtype=jnp.int32) % pad_mod
    return jnp.concatenate([row, pad]).reshape(NS * NCHUNKP, CS)


@jax.jit
def kernel(x, adj_t, adj_t2):
    xa = x[:, :DH]
    xb = x[:, DH:]
    d1 = _pad_idx(adj_t[0], GARB, ACC_ROWS - GARB)
    s1 = _pad_idx(adj_t[1], 0, N_NODES)
    d2 = _pad_idx(adj_t2[0], GARB, ACC_ROWS - GARB)
    s2 = _pad_idx(adj_t2[1], 0, N_NODES)
    o1a, o1b, o2a, o2b = _sc_kernel(xa, xb, d1, s1, d2, s2)
    return jnp.concatenate([o1a, o1b, o2a, o2b], axis=1)


# speedup vs baseline: 1.0019x; 1.0019x over previous
import functools

import jax
import jax.numpy as jnp
from jax import lax
from jax.experimental import pallas as pl
from jax.experimental.pallas import tpu as pltpu
from jax.experimental.pallas import tpu_sc as plsc

N_NODES = 10000
D = 128
DH = D // 2
E = 320000
NS = 16
CS = 128
NCHUNKP = 160
EPAD = NS * NCHUNKP * CS
GARB = N_NODES
ACC_ROWS = 10016
ZB = 16
WB = 80
NB = 4
NBI = 8


def _hop_pass(x_hbm, o_hbm, d_hbm, s_hbm, xspm, isd, iss, rbufs, zbuf, acc,
              isems, gsems, csems, s, row0):
    def xblk(j, carry):
        r0 = (s + j * NS) * WB
        pltpu.sync_copy(x_hbm.at[pl.ds(r0, WB)], xspm.at[pl.ds(r0, WB)])
        return carry

    lax.fori_loop(0, (140 - s) // 16, xblk, 0)

    def zblk(j, carry):
        pltpu.sync_copy(zbuf, acc.at[pl.ds((s + j * NS) * ZB, ZB)])
        return carry

    lax.fori_loop(0, (641 - s) // 16, zblk, 0)
    plsc.subcore_barrier()

    def fire_idx(i, t):
        pltpu.async_copy(d_hbm.at[row0 + i], isd.at[t], isems[t])
        pltpu.async_copy(s_hbm.at[row0 + i], iss.at[t], isems[t])

    def wait_idx(i, t):
        pltpu.make_async_copy(d_hbm.at[row0 + i], isd.at[t], isems[t]).wait()
        pltpu.make_async_copy(s_hbm.at[row0 + i], iss.at[t], isems[t]).wait()

    def fire_gather(i, b, t):
        pltpu.async_copy(xspm.at[iss.at[t]], rbufs[b], gsems[b])

    def wait_gather(i, b, t):
        pltpu.make_async_copy(xspm.at[iss.at[t]], rbufs[b], gsems[b]).wait()

    def fire_scatter(i, b, t):
        pltpu.async_copy(rbufs[b], acc.at[isd.at[t]], csems[b], add=True)

    def wait_scatter(i, b, t):
        pltpu.make_async_copy(rbufs[b], acc.at[isd.at[t]], csems[b]).wait()

    for i in range(4):
        fire_idx(i, i)
    wait_idx(0, 0)
    fire_gather(0, 0, 0)
    wait_idx(1, 1)
    fire_gather(1, 1, 1)

    def do_step(i, ring_i, ring_i2, slot_i, slot_i2, slot_i4,
                skip_scatter_wait):
        @pl.when(i + 2 < NCHUNKP)
        def _():
            if not skip_scatter_wait:
                wait_scatter(i - 2, ring_i2, (slot_i2 + NBI - 4) % NBI)
            wait_idx(i + 2, slot_i2)
            fire_gather(i + 2, ring_i2, slot_i2)

        @pl.when(i + 4 < NCHUNKP)
        def _():
            fire_idx(i + 4, slot_i4)

        wait_gather(i, ring_i, slot_i)
        fire_scatter(i, ring_i, slot_i)

    for b in range(NBI):
        do_step(b, b % NB, (b + 2) % NB, b % NBI, (b + 2) % NBI,
                (b + 4) % NBI, b < 2)

    def group(g, carry):
        i0 = NBI * g
        for b in range(NBI):
            i = i0 + b
            do_step(i, b % NB, (b + 2) % NB, b % NBI, (b + 2) % NBI,
                    (b + 4) % NBI, False)
        return carry

    lax.fori_loop(1, NCHUNKP // NBI, group, 0)

    for i in range(NCHUNKP - NB, NCHUNKP):
        wait_scatter(i, i % NB, i % NBI)
    plsc.subcore_barrier()

    def wblk(j, carry):
        r0 = (s + j * NS) * WB
        pltpu.sync_copy(acc.at[pl.ds(r0, WB)], o_hbm.at[pl.ds(r0, WB)])
        return carry

    lax.fori_loop(0, (140 - s) // 16, wblk, 0)
    plsc.subcore_barrier()


def _body(xa, xb, d1, s1, d2, s2, o1a, o1b, o2a, o2b,
          isd, iss, r0, r1, r2, r3, zbuf, xspm, acc,
          i0, i1, i2, i3, i4, i5, i6, i7,
          g0, g1, g2, g3, c0, c1, c2, c3):
    c = lax.axis_index("c")
    s = lax.axis_index("s")
    rbufs = (r0, r1, r2, r3)
    isems = (i0, i1, i2, i3, i4, i5, i6, i7)
    gsems = (g0, g1, g2, g3)
    csems = (c0, c1, c2, c3)
    row0 = s * NCHUNKP

    def zrow(i, carry):
        for k in range(DH // 16):
            zbuf[i, pl.ds(k * 16, 16)] = jnp.zeros((16,), jnp.float32)
        return carry

    lax.fori_loop(0, ZB, zrow, 0)

    @pl.when(c == 0)
    def _():
        _hop_pass(xa, o1a, d1, s1, xspm, isd, iss, rbufs, zbuf, acc,
                  isems, gsems, csems, s, row0)
        _hop_pass(xb, o1b, d1, s1, xspm, isd, iss, rbufs, zbuf, acc,
                  isems, gsems, csems, s, row0)

    @pl.when(c == 1)
    def _():
        _hop_pass(xa, o2a, d2, s2, xspm, isd, iss, rbufs, zbuf, acc,
                  isems, gsems, csems, s, row0)
        _hop_pass(xb, o2b, d2, s2, xspm, isd, iss, rbufs, zbuf, acc,
                  isems, gsems, csems, s, row0)


_half_out = jax.ShapeDtypeStruct((N_NODES, DH), jnp.float32)

_sc_kernel = functools.partial(
    pl.kernel,
    mesh=plsc.VectorSubcoreMesh(core_axis_name="c", subcore_axis_name="s"),
    out_type=[_half_out, _half_out, _half_out, _half_out],
    compiler_params=pltpu.CompilerParams(use_tc_tiling_on_sc=False),
    scratch_types=[
        pltpu.VMEM((NBI, CS), jnp.int32),
        pltpu.VMEM((NBI, CS), jnp.int32),
    ] + [pltpu.VMEM((CS, DH), jnp.float32)] * NB
    + [
        pltpu.VMEM((ZB, DH), jnp.float32),
        pltpu.VMEM_SHARED((N_NODES, DH), jnp.float32),
        pltpu.VMEM_SHARED((ACC_ROWS, DH), jnp.float32),
    ] + [pltpu.SemaphoreType.DMA] * (NBI + 2 * NB),
)(_body)


def _pad_idx(row, pad_base, pad_mod):
    pad = pad_base + jnp.arange(EPAD - E, dtype=jnp.int32) % pad_mod
    return jnp.concatenate([row, pad]).reshape(NS * NCHUNKP, CS)


@jax.jit
def kernel(x, adj_t, adj_t2):
    xa = x[:, :DH]
    xb = x[:, DH:]
    d1 = _pad_idx(adj_t[0], GARB, ACC_ROWS - GARB)
    s1 = _pad_idx(adj_t[1], 0, N_NODES)
    d2 = _pad_idx(adj_t2[0], GARB, ACC_ROWS - GARB)
    s2 = _pad_idx(adj_t2[1], 0, N_NODES)
    o1a, o1b, o2a, o2b = _sc_kernel(xa, xb, d1, s1, d2, s2)
    return jnp.concatenate([o1a, o1b, o2a, o2b], axis=1)
